# Initial kernel scaffold; baseline (speedup 1.0000x reference)
import jax
import jax.numpy as jnp
from jax.experimental import pallas as pl


def kernel(t, x, params, edge_rows, edge_cols, h_initial):
    raise NotImplementedError("write your pallas kernel here")


# speedup vs baseline: 6.6990x; 6.6990x over previous
import jax
import jax.numpy as jnp
from jax.experimental import pallas as pl
from jax.experimental.pallas import tpu as pltpu

N_PART = 22
N_DIM = 3
HIDDEN = 64
N_LAYERS = 5
COORDS_RANGE = 3.0
G_BLK = 8

_f32 = jnp.float32
_bf16 = jnp.bfloat16


def _silu(v):
    return v * jax.nn.sigmoid(v)


def _bdot(a, w):
    return jnp.dot(a.astype(_bf16), w.astype(_bf16),
                   preferred_element_type=_f32)


def _trunc(v):
    return v.astype(_bf16).astype(_f32)


def _egnn_body(t_ref, x_ref, hinit_ref,
               emb_wh_ref, emb_wt_ref, emb_b_ref,
               e0a_ref, e0b_ref, e0wr_ref, e0we_ref, e0bias_ref,
               e1w_ref, e1b_ref, attw_ref, attb_ref,
               c0w_ref, c0b_ref, c1w_ref,
               n0a_ref, n0b_ref, n0bias_ref, n1w_ref, n1b_ref,
               out_ref):
    G = G_BLK
    P = N_PART

    t_blk = t_ref[...]
    x0 = x_ref[...]

    base = _bdot(hinit_ref[...], emb_wh_ref[...]) + emb_b_ref[...]
    ht = _trunc(t_blk) * _trunc(emb_wt_ref[...])
    h = (base[None, :, :] + ht[:, None, :]).reshape(G * P, HIDDEN)

    diff0 = x0[:, :, None, :] - x0[:, None, :, :]
    ea4 = jnp.sum(diff0 * diff0, axis=-1, keepdims=True)
    ea4t = _trunc(ea4)

    ii = jax.lax.broadcasted_iota(jnp.int32, (1, P, P, 1), 1)
    jj = jax.lax.broadcasted_iota(jnp.int32, (1, P, P, 1), 2)
    mask = (ii != jj).astype(_f32)

    x = x0
    for l in range(N_LAYERS):
        if l == 0:
            diff, radial = diff0, ea4
        else:
            diff = x[:, :, None, :] - x[:, None, :, :]
            radial = jnp.sum(diff * diff, axis=-1, keepdims=True)

        hr = _bdot(h, e0a_ref[l])
        hc = _bdot(h, e0b_ref[l])
        pre = (hr.reshape(G, P, 1, HIDDEN) + hc.reshape(G, 1, P, HIDDEN)
               + _trunc(radial) * _trunc(e0wr_ref[l])
               + ea4t * _trunc(e0we_ref[l]) + e0bias_ref[l])
        m = _silu(pre).reshape(G * P * P, HIDDEN)
        m = _silu(_bdot(m, e1w_ref[l]) + e1b_ref[l])
        att = jax.nn.sigmoid(_bdot(m, attw_ref[l]) + attb_ref[l])
        att4 = att.reshape(G, P, P, 1) * mask
        m4 = m.reshape(G, P, P, HIDDEN) * att4
        agg = jnp.sum(m4, axis=2).reshape(G * P, HIDDEN)

        phi = _silu(_bdot(m4.reshape(G * P * P, HIDDEN), c0w_ref[l])
                    + c0b_ref[l])
        w_e = jnp.tanh(_bdot(phi, c1w_ref[l])) * COORDS_RANGE
        x = x + jnp.sum(diff * w_e.reshape(G, P, P, 1), axis=2)

        upd = _silu(_bdot(h, n0a_ref[l]) + _bdot(agg, n0b_ref[l])
                    + n0bias_ref[l])
        h = h + _bdot(upd, n1w_ref[l]) + n1b_ref[l]

    vel = x - x0
    vel = vel - jnp.mean(vel, axis=1, keepdims=True)
    out_ref[...] = vel


def kernel(t, x, params, edge_rows, edge_cols, h_initial):
    B = x.shape[0]
    x3 = x.reshape(B, N_PART, N_DIM)
    t2 = t.reshape(B, 1)

    layers = params["layers"]

    def stk(fn):
        return jnp.stack([fn(lp) for lp in layers])

    emb_w = params["emb"]["W"]
    nfeat = h_initial.shape[1]
    emb_wh = emb_w[:nfeat]
    emb_wt = emb_w[nfeat:nfeat + 1]
    emb_b = params["emb"]["b"].reshape(1, HIDDEN)

    e0a = stk(lambda p: p["e0"]["W"][:HIDDEN])
    e0b = stk(lambda p: p["e0"]["W"][HIDDEN:2 * HIDDEN])
    e0wr = stk(lambda p: p["e0"]["W"][2 * HIDDEN:2 * HIDDEN + 1])
    e0we = stk(lambda p: p["e0"]["W"][2 * HIDDEN + 1:2 * HIDDEN + 2])
    e0bias = stk(lambda p: p["e0"]["b"].reshape(1, HIDDEN))
    e1w = stk(lambda p: p["e1"]["W"])
    e1b = stk(lambda p: p["e1"]["b"].reshape(1, HIDDEN))
    attw = stk(lambda p: p["att"]["W"])
    attb = stk(lambda p: p["att"]["b"].reshape(1, 1))
    c0w = stk(lambda p: p["c0"]["W"])
    c0b = stk(lambda p: p["c0"]["b"].reshape(1, HIDDEN))
    c1w = stk(lambda p: p["c1"]["W"])
    n0a = stk(lambda p: p["n0"]["W"][:HIDDEN])
    n0b = stk(lambda p: p["n0"]["W"][HIDDEN:])
    n0bias = stk(lambda p: p["n0"]["b"].reshape(1, HIDDEN))
    n1w = stk(lambda p: p["n1"]["W"])
    n1b = stk(lambda p: p["n1"]["b"].reshape(1, HIDDEN))

    grid = (B // G_BLK,)
    full = lambda shp: pl.BlockSpec(shp, lambda b: (0,) * len(shp))

    in_specs = [
        pl.BlockSpec((G_BLK, 1), lambda b: (b, 0)),
        pl.BlockSpec((G_BLK, N_PART, N_DIM), lambda b: (b, 0, 0)),
        full(h_initial.shape),
        full(emb_wh.shape), full(emb_wt.shape), full(emb_b.shape),
        full(e0a.shape), full(e0b.shape), full(e0wr.shape),
        full(e0we.shape), full(e0bias.shape),
        full(e1w.shape), full(e1b.shape), full(attw.shape), full(attb.shape),
        full(c0w.shape), full(c0b.shape), full(c1w.shape),
        full(n0a.shape), full(n0b.shape), full(n0bias.shape),
        full(n1w.shape), full(n1b.shape),
    ]

    out = pl.pallas_call(
        _egnn_body,
        grid=grid,
        in_specs=in_specs,
        out_specs=pl.BlockSpec((G_BLK, N_PART, N_DIM), lambda b: (b, 0, 0)),
        out_shape=jax.ShapeDtypeStruct((B, N_PART, N_DIM), _f32),
        compiler_params=pltpu.CompilerParams(
            dimension_semantics=("arbitrary",)),
    )(t2, x3, h_initial,
      emb_wh, emb_wt, emb_b,
      e0a, e0b, e0wr, e0we, e0bias,
      e1w, e1b, attw, attb,
      c0w, c0b, c1w,
      n0a, n0b, n0bias, n1w, n1b)

    return out.reshape(B, N_PART * N_DIM)


# speedup vs baseline: 7.3779x; 1.1013x over previous
import jax
import jax.numpy as jnp
from jax.experimental import pallas as pl
from jax.experimental.pallas import tpu as pltpu

N_PART = 22
N_DIM = 3
HIDDEN = 64
N_LAYERS = 5
COORDS_RANGE = 3.0
G_BLK = 16

_f32 = jnp.float32
_bf16 = jnp.bfloat16


def _silu(v):
    return v * jax.nn.sigmoid(v)


def _bdot(a, w):
    return jnp.dot(a.astype(_bf16), w.astype(_bf16),
                   preferred_element_type=_f32)


def _trunc(v):
    return v.astype(_bf16).astype(_f32)


def _egnn_body(t_ref, x_ref, hinit_ref,
               emb_wh_ref, emb_wt_ref, emb_b_ref,
               e0a_ref, e0b_ref, e0wr_ref, e0we_ref, e0bias_ref,
               e1w_ref, e1b_ref, attw_ref, attb_ref,
               c0w_ref, c0b_ref, c1w_ref,
               n0a_ref, n0b_ref, n0bias_ref, n1w_ref, n1b_ref,
               out_ref):
    G = G_BLK
    P = N_PART

    t_blk = t_ref[...]
    x0 = x_ref[...]

    base = _bdot(hinit_ref[...], emb_wh_ref[...]) + emb_b_ref[...]
    ht = _trunc(t_blk) * _trunc(emb_wt_ref[...])
    h = (base[None, :, :] + ht[:, None, :]).reshape(G * P, HIDDEN)

    diff0 = x0[:, :, None, :] - x0[:, None, :, :]
    ea4 = jnp.sum(diff0 * diff0, axis=-1, keepdims=True)
    ea4t = _trunc(ea4)

    ii = jax.lax.broadcasted_iota(jnp.int32, (1, P, P, 1), 1)
    jj = jax.lax.broadcasted_iota(jnp.int32, (1, P, P, 1), 2)
    mask = (ii != jj).astype(_f32)

    x = x0
    for l in range(N_LAYERS):
        if l == 0:
            diff, radial = diff0, ea4
        else:
            diff = x[:, :, None, :] - x[:, None, :, :]
            radial = jnp.sum(diff * diff, axis=-1, keepdims=True)

        hr = _bdot(h, e0a_ref[l])
        hc = _bdot(h, e0b_ref[l])
        pre = (hr.reshape(G, P, 1, HIDDEN) + hc.reshape(G, 1, P, HIDDEN)
               + _trunc(radial) * _trunc(e0wr_ref[l])
               + ea4t * _trunc(e0we_ref[l]) + e0bias_ref[l])
        m = _silu(pre).reshape(G * P * P, HIDDEN)
        m = _silu(_bdot(m, e1w_ref[l]) + e1b_ref[l])
        att = jax.nn.sigmoid(_bdot(m, attw_ref[l]) + attb_ref[l])
        att4 = att.reshape(G, P, P, 1) * mask
        m4 = m.reshape(G, P, P, HIDDEN) * att4
        agg = jnp.sum(m4, axis=2).reshape(G * P, HIDDEN)

        phi = _silu(_bdot(m4.reshape(G * P * P, HIDDEN), c0w_ref[l])
                    + c0b_ref[l])
        w_e = jnp.tanh(_bdot(phi, c1w_ref[l])) * COORDS_RANGE
        x = x + jnp.sum(diff * w_e.reshape(G, P, P, 1), axis=2)

        upd = _silu(_bdot(h, n0a_ref[l]) + _bdot(agg, n0b_ref[l])
                    + n0bias_ref[l])
        h = h + _bdot(upd, n1w_ref[l]) + n1b_ref[l]

    vel = x - x0
    vel = vel - jnp.mean(vel, axis=1, keepdims=True)
    out_ref[...] = vel


def kernel(t, x, params, edge_rows, edge_cols, h_initial):
    B = x.shape[0]
    x3 = x.reshape(B, N_PART, N_DIM)
    t2 = t.reshape(B, 1)

    layers = params["layers"]

    def stk(fn):
        return jnp.stack([fn(lp) for lp in layers])

    emb_w = params["emb"]["W"]
    nfeat = h_initial.shape[1]
    emb_wh = emb_w[:nfeat]
    emb_wt = emb_w[nfeat:nfeat + 1]
    emb_b = params["emb"]["b"].reshape(1, HIDDEN)

    e0a = stk(lambda p: p["e0"]["W"][:HIDDEN])
    e0b = stk(lambda p: p["e0"]["W"][HIDDEN:2 * HIDDEN])
    e0wr = stk(lambda p: p["e0"]["W"][2 * HIDDEN:2 * HIDDEN + 1])
    e0we = stk(lambda p: p["e0"]["W"][2 * HIDDEN + 1:2 * HIDDEN + 2])
    e0bias = stk(lambda p: p["e0"]["b"].reshape(1, HIDDEN))
    e1w = stk(lambda p: p["e1"]["W"])
    e1b = stk(lambda p: p["e1"]["b"].reshape(1, HIDDEN))
    attw = stk(lambda p: p["att"]["W"])
    attb = stk(lambda p: p["att"]["b"].reshape(1, 1))
    c0w = stk(lambda p: p["c0"]["W"])
    c0b = stk(lambda p: p["c0"]["b"].reshape(1, HIDDEN))
    c1w = stk(lambda p: p["c1"]["W"])
    n0a = stk(lambda p: p["n0"]["W"][:HIDDEN])
    n0b = stk(lambda p: p["n0"]["W"][HIDDEN:])
    n0bias = stk(lambda p: p["n0"]["b"].reshape(1, HIDDEN))
    n1w = stk(lambda p: p["n1"]["W"])
    n1b = stk(lambda p: p["n1"]["b"].reshape(1, HIDDEN))

    grid = (B // G_BLK,)
    full = lambda shp: pl.BlockSpec(shp, lambda b: (0,) * len(shp))

    in_specs = [
        pl.BlockSpec((G_BLK, 1), lambda b: (b, 0)),
        pl.BlockSpec((G_BLK, N_PART, N_DIM), lambda b: (b, 0, 0)),
        full(h_initial.shape),
        full(emb_wh.shape), full(emb_wt.shape), full(emb_b.shape),
        full(e0a.shape), full(e0b.shape), full(e0wr.shape),
        full(e0we.shape), full(e0bias.shape),
        full(e1w.shape), full(e1b.shape), full(attw.shape), full(attb.shape),
        full(c0w.shape), full(c0b.shape), full(c1w.shape),
        full(n0a.shape), full(n0b.shape), full(n0bias.shape),
        full(n1w.shape), full(n1b.shape),
    ]

    out = pl.pallas_call(
        _egnn_body,
        grid=grid,
        in_specs=in_specs,
        out_specs=pl.BlockSpec((G_BLK, N_PART, N_DIM), lambda b: (b, 0, 0)),
        out_shape=jax.ShapeDtypeStruct((B, N_PART, N_DIM), _f32),
        compiler_params=pltpu.CompilerParams(
            dimension_semantics=("arbitrary",)),
    )(t2, x3, h_initial,
      emb_wh, emb_wt, emb_b,
      e0a, e0b, e0wr, e0we, e0bias,
      e1w, e1b, attw, attb,
      c0w, c0b, c1w,
      n0a, n0b, n0bias, n1w, n1b)

    return out.reshape(B, N_PART * N_DIM)


# speedup vs baseline: 11.8889x; 1.6114x over previous
import jax
import jax.numpy as jnp
from jax.experimental import pallas as pl
from jax.experimental.pallas import tpu as pltpu

N_PART = 22
PN = 24
N_DIM = 3
HIDDEN = 64
N_LAYERS = 5
COORDS_RANGE = 3.0
G_BLK = 16

_f32 = jnp.float32
_bf16 = jnp.bfloat16


def _silu(v):
    return v * jax.nn.sigmoid(v)


def _bdot(a, w):
    return jnp.dot(a.astype(_bf16), w.astype(_bf16),
                   preferred_element_type=_f32)


def _egnn_body(t_ref, x_ref, hinit_ref,
               emb_wh_ref, emb_wt_ref, emb_b_ref,
               e0a_ref, e0b_ref, e0wr_ref, e0we_ref, e0bias_ref,
               e1w_ref, e1b_ref, attw_ref, attb_ref,
               c0w_ref, c0b_ref, c1w_ref,
               n0a_ref, n0b_ref, n0bias_ref, n1w_ref, n1b_ref,
               out_ref):
    G = G_BLK
    P = PN
    E = G * P * P

    t_blk = t_ref[...]
    x0 = x_ref[...]

    base = _bdot(hinit_ref[...], emb_wh_ref[...]) + emb_b_ref[...]
    tb = t_blk.astype(_bf16).astype(_f32)
    ht = tb * emb_wt_ref[...].astype(_bf16).astype(_f32)
    h = (base[None, :, :] + ht[:, None, :]).reshape(G * P, HIDDEN)

    ii = jax.lax.broadcasted_iota(jnp.int32, (1, P, P, 1), 1)
    jj = jax.lax.broadcasted_iota(jnp.int32, (1, P, P, 1), 2)
    valid = jnp.logical_and(jnp.logical_and(ii < N_PART, jj < N_PART),
                            ii != jj)
    mask_b = jnp.broadcast_to(valid, (1, P, P, HIDDEN)).astype(_f32)
    mask_3 = jnp.broadcast_to(valid, (1, P, P, N_DIM)).astype(_f32)
    nmask = (jax.lax.broadcasted_iota(jnp.int32, (1, P, 1), 1)
             < N_PART).astype(_f32)

    diff0 = x0[:, :, None, :] - x0[:, None, :, :]
    ea = jnp.sum(diff0 * diff0, axis=-1,
                 keepdims=True).reshape(E, 1).astype(_bf16)

    x = x0
    for l in range(N_LAYERS):
        if l == 0:
            diff = diff0
        else:
            diff = x[:, :, None, :] - x[:, None, :, :]
        radial = jnp.sum(diff * diff, axis=-1, keepdims=True).reshape(E, 1)

        hr = _bdot(h, e0a_ref[l])
        hc = _bdot(h, e0b_ref[l])
        scal = (jnp.dot(radial.astype(_bf16), e0wr_ref[l].astype(_bf16),
                        preferred_element_type=_f32)
                + jnp.dot(ea, e0we_ref[l].astype(_bf16),
                          preferred_element_type=_f32))
        pre = (hr.reshape(G, P, 1, HIDDEN) + hc.reshape(G, 1, P, HIDDEN)
               + scal.reshape(G, P, P, HIDDEN) + e0bias_ref[l])
        m = _silu(pre.reshape(E, HIDDEN))
        m = _silu(_bdot(m, e1w_ref[l]) + e1b_ref[l])
        att = jax.nn.sigmoid(_bdot(m, attw_ref[l]) + attb_ref[l])
        m4 = (m * att).reshape(G, P, P, HIDDEN) * mask_b
        agg = jnp.sum(m4, axis=2).reshape(G * P, HIDDEN)

        phi = _silu(_bdot(m4.reshape(E, HIDDEN), c0w_ref[l]) + c0b_ref[l])
        w_e = jnp.tanh(_bdot(phi, c1w_ref[l])) * COORDS_RANGE
        trans = diff * (w_e.reshape(G, P, P, N_DIM) * mask_3)
        x = x + jnp.sum(trans, axis=2)

        upd = _silu(_bdot(h, n0a_ref[l]) + _bdot(agg, n0b_ref[l])
                    + n0bias_ref[l])
        h = h + _bdot(upd, n1w_ref[l]) + n1b_ref[l]

    vel = (x - x0) * nmask
    vel = vel - jnp.sum(vel, axis=1, keepdims=True) * (1.0 / N_PART)
    out_ref[...] = vel


def kernel(t, x, params, edge_rows, edge_cols, h_initial):
    B = x.shape[0]
    x3 = jnp.pad(x.reshape(B, N_PART, N_DIM),
                 ((0, 0), (0, PN - N_PART), (0, 0)))
    t2 = t.reshape(B, 1)
    hinit_p = jnp.pad(h_initial, ((0, PN - N_PART), (0, 0)))

    layers = params["layers"]

    def stk(fn):
        return jnp.stack([fn(lp) for lp in layers])

    emb_w = params["emb"]["W"]
    nfeat = h_initial.shape[1]
    emb_wh = emb_w[:nfeat]
    emb_wt = emb_w[nfeat:nfeat + 1]
    emb_b = params["emb"]["b"].reshape(1, HIDDEN)

    e0a = stk(lambda p: p["e0"]["W"][:HIDDEN])
    e0b = stk(lambda p: p["e0"]["W"][HIDDEN:2 * HIDDEN])
    e0wr = stk(lambda p: p["e0"]["W"][2 * HIDDEN:2 * HIDDEN + 1])
    e0we = stk(lambda p: p["e0"]["W"][2 * HIDDEN + 1:2 * HIDDEN + 2])
    e0bias = stk(lambda p: p["e0"]["b"].reshape(1, HIDDEN))
    e1w = stk(lambda p: p["e1"]["W"])
    e1b = stk(lambda p: p["e1"]["b"].reshape(1, HIDDEN))
    attw = stk(lambda p: jnp.broadcast_to(p["att"]["W"], (HIDDEN, HIDDEN)))
    attb = stk(lambda p: p["att"]["b"].reshape(1, 1))
    c0w = stk(lambda p: p["c0"]["W"])
    c0b = stk(lambda p: p["c0"]["b"].reshape(1, HIDDEN))
    c1w = stk(lambda p: jnp.broadcast_to(p["c1"]["W"], (HIDDEN, N_DIM)))
    n0a = stk(lambda p: p["n0"]["W"][:HIDDEN])
    n0b = stk(lambda p: p["n0"]["W"][HIDDEN:])
    n0bias = stk(lambda p: p["n0"]["b"].reshape(1, HIDDEN))
    n1w = stk(lambda p: p["n1"]["W"])
    n1b = stk(lambda p: p["n1"]["b"].reshape(1, HIDDEN))

    grid = (B // G_BLK,)
    full = lambda shp: pl.BlockSpec(shp, lambda b: (0,) * len(shp))

    in_specs = [
        pl.BlockSpec((G_BLK, 1), lambda b: (b, 0)),
        pl.BlockSpec((G_BLK, PN, N_DIM), lambda b: (b, 0, 0)),
        full(hinit_p.shape),
        full(emb_wh.shape), full(emb_wt.shape), full(emb_b.shape),
        full(e0a.shape), full(e0b.shape), full(e0wr.shape),
        full(e0we.shape), full(e0bias.shape),
        full(e1w.shape), full(e1b.shape), full(attw.shape), full(attb.shape),
        full(c0w.shape), full(c0b.shape), full(c1w.shape),
        full(n0a.shape), full(n0b.shape), full(n0bias.shape),
        full(n1w.shape), full(n1b.shape),
    ]

    out = pl.pallas_call(
        _egnn_body,
        grid=grid,
        in_specs=in_specs,
        out_specs=pl.BlockSpec((G_BLK, PN, N_DIM), lambda b: (b, 0, 0)),
        out_shape=jax.ShapeDtypeStruct((B, PN, N_DIM), _f32),
        compiler_params=pltpu.CompilerParams(
            dimension_semantics=("arbitrary",)),
    )(t2, x3, hinit_p,
      emb_wh, emb_wt, emb_b,
      e0a, e0b, e0wr, e0we, e0bias,
      e1w, e1b, attw, attb,
      c0w, c0b, c1w,
      n0a, n0b, n0bias, n1w, n1b)

    return out[:, :N_PART, :].reshape(B, N_PART * N_DIM)


# speedup vs baseline: 13.0213x; 1.0952x over previous
import jax
import jax.numpy as jnp
from jax.experimental import pallas as pl
from jax.experimental.pallas import tpu as pltpu

N_PART = 22
PN = 24
N_DIM = 3
HIDDEN = 64
N_LAYERS = 5
COORDS_RANGE = 3.0
G_BLK = 32

_f32 = jnp.float32
_bf16 = jnp.bfloat16


def _silu(v):
    return v * jax.nn.sigmoid(v)


def _bdot(a, w):
    return jnp.dot(a.astype(_bf16), w.astype(_bf16),
                   preferred_element_type=_f32)


def _egnn_body(t_ref, x_ref, hinit_ref,
               emb_wh_ref, emb_wt_ref, emb_b_ref,
               e0a_ref, e0b_ref, e0wr_ref, e0we_ref, e0bias_ref,
               e1w_ref, e1b_ref, attw_ref, attb_ref,
               c0w_ref, c0b_ref, c1w_ref,
               n0a_ref, n0b_ref, n0bias_ref, n1w_ref, n1b_ref,
               out_ref):
    G = G_BLK
    P = PN
    E = G * P * P

    t_blk = t_ref[...]
    x0 = x_ref[...]

    base = _bdot(hinit_ref[...], emb_wh_ref[...]) + emb_b_ref[...]
    tb = t_blk.astype(_bf16).astype(_f32)
    ht = tb * emb_wt_ref[...].astype(_bf16).astype(_f32)
    h = (base[None, :, :] + ht[:, None, :]).reshape(G * P, HIDDEN)

    ii = jax.lax.broadcasted_iota(jnp.int32, (1, P, P, 1), 1)
    jj = jax.lax.broadcasted_iota(jnp.int32, (1, P, P, 1), 2)
    valid = jnp.logical_and(jnp.logical_and(ii < N_PART, jj < N_PART),
                            ii != jj)
    mask_b = jnp.broadcast_to(valid, (1, P, P, HIDDEN)).astype(_f32)
    mask_3 = jnp.broadcast_to(valid, (1, P, P, N_DIM)).astype(_f32)
    nmask = (jax.lax.broadcasted_iota(jnp.int32, (1, P, 1), 1)
             < N_PART).astype(_f32)

    diff0 = x0[:, :, None, :] - x0[:, None, :, :]
    ea = jnp.sum(diff0 * diff0, axis=-1,
                 keepdims=True).reshape(E, 1).astype(_bf16)

    x = x0
    for l in range(N_LAYERS):
        if l == 0:
            diff = diff0
        else:
            diff = x[:, :, None, :] - x[:, None, :, :]
        radial = jnp.sum(diff * diff, axis=-1, keepdims=True).reshape(E, 1)

        hr = _bdot(h, e0a_ref[l])
        hc = _bdot(h, e0b_ref[l])
        scal = (jnp.dot(radial.astype(_bf16), e0wr_ref[l].astype(_bf16),
                        preferred_element_type=_f32)
                + jnp.dot(ea, e0we_ref[l].astype(_bf16),
                          preferred_element_type=_f32))
        pre = (hr.reshape(G, P, 1, HIDDEN) + hc.reshape(G, 1, P, HIDDEN)
               + scal.reshape(G, P, P, HIDDEN) + e0bias_ref[l])
        m = _silu(pre.reshape(E, HIDDEN))
        m = _silu(_bdot(m, e1w_ref[l]) + e1b_ref[l])
        att = jax.nn.sigmoid(_bdot(m, attw_ref[l]) + attb_ref[l])
        m4 = (m * att).reshape(G, P, P, HIDDEN) * mask_b
        agg = jnp.sum(m4, axis=2).reshape(G * P, HIDDEN)

        phi = _silu(_bdot(m4.reshape(E, HIDDEN), c0w_ref[l]) + c0b_ref[l])
        w_e = jnp.tanh(_bdot(phi, c1w_ref[l])) * COORDS_RANGE
        trans = diff * (w_e.reshape(G, P, P, N_DIM) * mask_3)
        x = x + jnp.sum(trans, axis=2)

        upd = _silu(_bdot(h, n0a_ref[l]) + _bdot(agg, n0b_ref[l])
                    + n0bias_ref[l])
        h = h + _bdot(upd, n1w_ref[l]) + n1b_ref[l]

    vel = (x - x0) * nmask
    vel = vel - jnp.sum(vel, axis=1, keepdims=True) * (1.0 / N_PART)
    out_ref[...] = vel


def kernel(t, x, params, edge_rows, edge_cols, h_initial):
    B = x.shape[0]
    x3 = jnp.pad(x.reshape(B, N_PART, N_DIM),
                 ((0, 0), (0, PN - N_PART), (0, 0)))
    t2 = t.reshape(B, 1)
    hinit_p = jnp.pad(h_initial, ((0, PN - N_PART), (0, 0)))

    layers = params["layers"]

    def stk(fn):
        return jnp.stack([fn(lp) for lp in layers])

    emb_w = params["emb"]["W"]
    nfeat = h_initial.shape[1]
    emb_wh = emb_w[:nfeat]
    emb_wt = emb_w[nfeat:nfeat + 1]
    emb_b = params["emb"]["b"].reshape(1, HIDDEN)

    e0a = stk(lambda p: p["e0"]["W"][:HIDDEN])
    e0b = stk(lambda p: p["e0"]["W"][HIDDEN:2 * HIDDEN])
    e0wr = stk(lambda p: p["e0"]["W"][2 * HIDDEN:2 * HIDDEN + 1])
    e0we = stk(lambda p: p["e0"]["W"][2 * HIDDEN + 1:2 * HIDDEN + 2])
    e0bias = stk(lambda p: p["e0"]["b"].reshape(1, HIDDEN))
    e1w = stk(lambda p: p["e1"]["W"])
    e1b = stk(lambda p: p["e1"]["b"].reshape(1, HIDDEN))
    attw = stk(lambda p: jnp.broadcast_to(p["att"]["W"], (HIDDEN, HIDDEN)))
    attb = stk(lambda p: p["att"]["b"].reshape(1, 1))
    c0w = stk(lambda p: p["c0"]["W"])
    c0b = stk(lambda p: p["c0"]["b"].reshape(1, HIDDEN))
    c1w = stk(lambda p: jnp.broadcast_to(p["c1"]["W"], (HIDDEN, N_DIM)))
    n0a = stk(lambda p: p["n0"]["W"][:HIDDEN])
    n0b = stk(lambda p: p["n0"]["W"][HIDDEN:])
    n0bias = stk(lambda p: p["n0"]["b"].reshape(1, HIDDEN))
    n1w = stk(lambda p: p["n1"]["W"])
    n1b = stk(lambda p: p["n1"]["b"].reshape(1, HIDDEN))

    grid = (B // G_BLK,)
    full = lambda shp: pl.BlockSpec(shp, lambda b: (0,) * len(shp))

    in_specs = [
        pl.BlockSpec((G_BLK, 1), lambda b: (b, 0)),
        pl.BlockSpec((G_BLK, PN, N_DIM), lambda b: (b, 0, 0)),
        full(hinit_p.shape),
        full(emb_wh.shape), full(emb_wt.shape), full(emb_b.shape),
        full(e0a.shape), full(e0b.shape), full(e0wr.shape),
        full(e0we.shape), full(e0bias.shape),
        full(e1w.shape), full(e1b.shape), full(attw.shape), full(attb.shape),
        full(c0w.shape), full(c0b.shape), full(c1w.shape),
        full(n0a.shape), full(n0b.shape), full(n0bias.shape),
        full(n1w.shape), full(n1b.shape),
    ]

    out = pl.pallas_call(
        _egnn_body,
        grid=grid,
        in_specs=in_specs,
        out_specs=pl.BlockSpec((G_BLK, PN, N_DIM), lambda b: (b, 0, 0)),
        out_shape=jax.ShapeDtypeStruct((B, PN, N_DIM), _f32),
        compiler_params=pltpu.CompilerParams(
            dimension_semantics=("arbitrary",)),
    )(t2, x3, hinit_p,
      emb_wh, emb_wt, emb_b,
      e0a, e0b, e0wr, e0we, e0bias,
      e1w, e1b, attw, attb,
      c0w, c0b, c1w,
      n0a, n0b, n0bias, n1w, n1b)

    return out[:, :N_PART, :].reshape(B, N_PART * N_DIM)
